# baseline (speedup 1.0000x reference)
import jax
import jax.numpy as jnp
from jax import lax
from jax.experimental import pallas as pl
from jax.experimental.pallas import tpu as pltpu
from jax.experimental.pallas import tpu_sc as plsc

_NC, _NS = 2, 16
_NW = _NC * _NS

_B, _H, _D, _V = 16384, 50, 32, 1000000
_SPW = _B // _NW
_LT = _SPW // 128
_DR = _D // 8


def _make_gather():
    mesh = plsc.VectorSubcoreMesh(
        core_axis_name="c", subcore_axis_name="s",
        num_cores=_NC, num_subcores=_NS,
    )

    def body(el_hbm, table_hbm, out_hbm, idx_v, idx_h, rows, tiles,
             g0, g1, w0, w1):
        gsem = (g0, g1)
        wsem = (w0, w1)
        wid = lax.axis_index("s") * _NC + lax.axis_index("c")
        base = wid * _SPW
        pltpu.sync_copy(el_hbm.at[pl.ds(base, _SPW)], idx_v)
        lanes = lax.iota(jnp.int32, 16)

        def build_idx(h, b):
            hvec = lanes * 0 + h
            for k in range(_SPW // 16):
                rowv = k * 16 + lanes
                idx_h[b, pl.ds(k * 16, 16)] = plsc.load_gather(
                    idx_v, [rowv, hvec]) * 4

        def gather_desc(b):
            return pltpu.make_async_copy(
                table_hbm.at[idx_h.at[b]], rows.at[b], gsem[b])

        def start_gather(b):
            pltpu.async_copy(table_hbm.at[idx_h.at[b]], rows.at[b], gsem[b])

        def out_slab(h):
            return out_hbm.at[pl.ds(h, 1), pl.ds(0, _DR), pl.ds(wid * _LT, _LT)]

        def tiles_view(b):
            return tiles.at[b, pl.ds(0, 1), pl.ds(0, _DR), pl.ds(0, _LT),
                            pl.ds(0, 8), pl.ds(0, 128)]

        def write_desc(b, h):
            return pltpu.make_async_copy(out_slab(h), tiles_view(b), wsem[b])

        dr_lo = lanes >> 3
        dr_hi = dr_lo + 2
        s_vec = lanes & 7
        zero16 = lanes * 0

        def transpose(b):
            def tj(jb, carry):
                for jj in range(16):
                    j = jb * 16 + jj
                    q_vec = zero16 + (j >> 7)
                    l_vec = zero16 + (j & 127)
                    v0 = rows[b, j, pl.ds(0, 16)]
                    v1 = rows[b, j, pl.ds(16, 16)]
                    plsc.store_scatter(
                        tiles.at[b, 0], [dr_lo, q_vec, s_vec, l_vec], v0)
                    plsc.store_scatter(
                        tiles.at[b, 0], [dr_hi, q_vec, s_vec, l_vec], v1)
                return carry
            lax.fori_loop(0, _SPW // 16, tj, 0)

        build_idx(0, 0)
        start_gather(0)
        build_idx(1, 1)
        start_gather(1)

        def step(hh, carry):
            for b in range(2):
                h = 2 * hh + b
                gather_desc(b).wait()

                @pl.when(hh > 0)
                def _():
                    write_desc(b, h - 2).wait()

                transpose(b)
                pltpu.async_copy(tiles_view(b), out_slab(h), wsem[b])

                @pl.when(h + 2 < _H)
                def _():
                    build_idx(h + 2, b)
                    start_gather(b)
            return carry

        lax.fori_loop(0, _H // 2, step, 0)
        write_desc(0, _H - 2).wait()
        write_desc(1, _H - 1).wait()

    return pl.kernel(
        body,
        out_type=jax.ShapeDtypeStruct((_H, _DR, 128, 8, 128), jnp.float32),
        mesh=mesh,
        scratch_types=[
            pltpu.VMEM((_SPW, _H), jnp.int32),
            pltpu.VMEM((2, _SPW), jnp.int32),
            pltpu.VMEM((2, _SPW, _D), jnp.float32),
            pltpu.VMEM((2, 1, _DR, _LT, 10, 129), jnp.float32),
            pltpu.SemaphoreType.DMA,
            pltpu.SemaphoreType.DMA,
            pltpu.SemaphoreType.DMA,
            pltpu.SemaphoreType.DMA,
        ],
        compiler_params=pltpu.CompilerParams(use_tc_tiling_on_sc=False, needs_layout_passes=False),
    )


def _tc_relayout(table):
    tt = table.T
    VB = 8192
    LAST = (_V - VB) // 128 * 128
    TCOL = _V // 128 * 128

    def body(x_hbm, o_hbm, xv, ov, xt, ot, sin, sout):
        g = pl.program_id(0)
        col = jnp.minimum(g * VB, LAST)
        cp = pltpu.make_async_copy(x_hbm.at[:, pl.ds(col, VB)], xv, sin)
        cp.start()
        cp.wait()
        ov[:, :, pl.ds(0, _D)] = xv[...].T.reshape(VB // 8, 8, _D)
        cpo = pltpu.make_async_copy(
            ov, o_hbm.at[pl.ds(col // 8, VB // 8)], sout)
        cpo.start()
        cpo.wait()

        @pl.when(g == 122)
        def _():
            ct = pltpu.make_async_copy(x_hbm.at[:, pl.ds(TCOL, 64)], xt, sin)
            ct.start()
            ct.wait()
            ot[:, :, pl.ds(0, _D)] = xt[...].T.reshape(8, 8, _D)
            cto = pltpu.make_async_copy(
                ot, o_hbm.at[pl.ds(TCOL // 8, 8)], sout)
            cto.start()
            cto.wait()

    out = pl.pallas_call(
        body,
        grid=(123,),
        in_specs=[pl.BlockSpec(memory_space=pltpu.MemorySpace.HBM)],
        out_specs=pl.BlockSpec(memory_space=pltpu.MemorySpace.HBM),
        out_shape=jax.ShapeDtypeStruct((_V // 8, 8, 128), jnp.float32),
        scratch_shapes=[pltpu.VMEM((32, VB), jnp.float32),
                        pltpu.VMEM((VB // 8, 8, 128), jnp.float32),
                        pltpu.VMEM((32, 64), jnp.float32),
                        pltpu.VMEM((8, 8, 128), jnp.float32),
                        pltpu.SemaphoreType.DMA,
                        pltpu.SemaphoreType.DMA],
    )(tt)
    return out.reshape(4 * _V, _D)


def kernel(element, table):
    out5 = _make_gather()(element, _tc_relayout(table))
    return out5.transpose((2, 4, 0, 1, 3)).reshape(_B, _H, _D)


# speedup vs baseline: 1.5532x; 1.5532x over previous
import jax
import jax.numpy as jnp
from jax import lax
from jax.experimental import pallas as pl
from jax.experimental.pallas import tpu as pltpu
from jax.experimental.pallas import tpu_sc as plsc

_NC, _NS = 2, 16
_NW = _NC * _NS

_B, _H, _D, _V = 16384, 50, 32, 1000000
_SPW = _B // _NW
_LT = _SPW // 128
_DR = _D // 8


def _make_gather():
    mesh = plsc.VectorSubcoreMesh(
        core_axis_name="c", subcore_axis_name="s",
        num_cores=_NC, num_subcores=_NS,
    )

    def body(el_hbm, table_hbm, out_hbm, idx_v, idx_h, rows, tiles,
             g0, g1, w0, w1):
        gsem = (g0, g1)
        wsem = (w0, w1)
        wid = lax.axis_index("s") * _NC + lax.axis_index("c")
        base = wid * _SPW
        pltpu.sync_copy(el_hbm.at[pl.ds(base, _SPW)], idx_v)
        lanes = lax.iota(jnp.int32, 16)

        def build_idx(h, b):
            hvec = lanes * 0 + h
            for k in range(_SPW // 16):
                rowv = k * 16 + lanes
                idx_h[b, pl.ds(k * 16, 16)] = plsc.load_gather(
                    idx_v, [rowv, hvec]) * 4

        def gather_desc(b):
            return pltpu.make_async_copy(
                table_hbm.at[idx_h.at[b]], rows.at[b], gsem[b])

        def start_gather(b):
            pltpu.async_copy(table_hbm.at[idx_h.at[b]], rows.at[b], gsem[b])

        def out_slab(h):
            return out_hbm.at[pl.ds(h, 1), pl.ds(0, _DR), pl.ds(wid * _LT, _LT)]

        def tiles_view(b):
            return tiles.at[b, pl.ds(0, 1), pl.ds(0, _DR), pl.ds(0, _LT),
                            pl.ds(0, 8), pl.ds(0, 128)]

        def write_desc(b, h):
            return pltpu.make_async_copy(out_slab(h), tiles_view(b), wsem[b])

        dr_lo = lanes >> 3
        dr_hi = dr_lo + 2
        s_vec = lanes & 7
        zero16 = lanes * 0

        def transpose(b):
            def tj(jb, carry):
                for jj in range(16):
                    j = jb * 16 + jj
                    q_vec = zero16 + (j >> 7)
                    l_vec = zero16 + (j & 127)
                    v0 = rows[b, j, pl.ds(0, 16)]
                    v1 = rows[b, j, pl.ds(16, 16)]
                    plsc.store_scatter(
                        tiles.at[b, 0], [dr_lo, q_vec, s_vec, l_vec], v0)
                    plsc.store_scatter(
                        tiles.at[b, 0], [dr_hi, q_vec, s_vec, l_vec], v1)
                return carry
            lax.fori_loop(0, _SPW // 16, tj, 0)

        build_idx(0, 0)
        start_gather(0)
        build_idx(1, 1)
        start_gather(1)

        def step(hh, carry):
            for b in range(2):
                h = 2 * hh + b
                gather_desc(b).wait()

                @pl.when(hh > 0)
                def _():
                    write_desc(b, h - 2).wait()

                transpose(b)
                pltpu.async_copy(tiles_view(b), out_slab(h), wsem[b])

                @pl.when(h + 2 < _H)
                def _():
                    build_idx(h + 2, b)
                    start_gather(b)
            return carry

        lax.fori_loop(0, _H // 2, step, 0)
        write_desc(0, _H - 2).wait()
        write_desc(1, _H - 1).wait()

    return pl.kernel(
        body,
        out_type=jax.ShapeDtypeStruct((_H, _DR, 128, 8, 128), jnp.float32),
        mesh=mesh,
        scratch_types=[
            pltpu.VMEM((_SPW, _H), jnp.int32),
            pltpu.VMEM((2, _SPW), jnp.int32),
            pltpu.VMEM((2, _SPW, _D), jnp.float32),
            pltpu.VMEM((2, 1, _DR, _LT, 10, 129), jnp.float32),
            pltpu.SemaphoreType.DMA,
            pltpu.SemaphoreType.DMA,
            pltpu.SemaphoreType.DMA,
            pltpu.SemaphoreType.DMA,
        ],
        compiler_params=pltpu.CompilerParams(use_tc_tiling_on_sc=False, needs_layout_passes=False),
    )


def _tc_relayout(table):
    tt = table.T
    VB = 8192
    NBLK = 123
    NG = 62
    LAST = (_V - VB) // 128 * 128
    TCOL = _V // 128 * 128

    def body(x_hbm, o_hbm, xv0, xv1, ov0, ov1, xt, ot, si0, si1, so0, so1):
        g = pl.program_id(0)
        xv = (xv0, xv1)
        ov = (ov0, ov1)
        si = (si0, si1)
        so = (so0, so1)

        def col_of(i):
            return jnp.minimum(i * VB, LAST)

        def rd(i, k):
            return pltpu.make_async_copy(
                x_hbm.at[:, pl.ds(col_of(i), VB)], xv[k], si[k])

        def wr(i, k):
            return pltpu.make_async_copy(
                ov[k], o_hbm.at[pl.ds(col_of(i) // 8, VB // 8)], so[k])

        for k in range(2):
            rd(2 * g + k, k).start()
        for k in range(2):
            i = 2 * g + k
            rd(i, k).wait()

            @pl.when(g > 0)
            def _():
                wr(i - 2, k).wait()

            ov[k][:, :, pl.ds(0, _D)] = xv[k][...].T.reshape(VB // 8, 8, _D)
            wr(i, k).start()

        @pl.when(g == NG - 1)
        def _():
            for k in range(2):
                wr(2 * g + k, k).wait()
            ct = pltpu.make_async_copy(x_hbm.at[:, pl.ds(TCOL, 64)], xt, si0)
            ct.start()
            ct.wait()
            ot[:, :, pl.ds(0, _D)] = xt[...].T.reshape(8, 8, _D)
            cto = pltpu.make_async_copy(
                ot, o_hbm.at[pl.ds(TCOL // 8, 8)], so0)
            cto.start()
            cto.wait()

    out = pl.pallas_call(
        body,
        grid=(NG,),
        in_specs=[pl.BlockSpec(memory_space=pltpu.MemorySpace.HBM)],
        out_specs=pl.BlockSpec(memory_space=pltpu.MemorySpace.HBM),
        out_shape=jax.ShapeDtypeStruct((_V // 8, 8, 128), jnp.float32),
        scratch_shapes=[pltpu.VMEM((32, VB), jnp.float32),
                        pltpu.VMEM((32, VB), jnp.float32),
                        pltpu.VMEM((VB // 8, 8, 128), jnp.float32),
                        pltpu.VMEM((VB // 8, 8, 128), jnp.float32),
                        pltpu.VMEM((32, 64), jnp.float32),
                        pltpu.VMEM((8, 8, 128), jnp.float32),
                        pltpu.SemaphoreType.DMA,
                        pltpu.SemaphoreType.DMA,
                        pltpu.SemaphoreType.DMA,
                        pltpu.SemaphoreType.DMA],
    )(tt)
    return out.reshape(4 * _V, _D)


def kernel(element, table):
    out5 = _make_gather()(element, _tc_relayout(table))
    return out5.transpose((2, 4, 0, 1, 3)).reshape(_B, _H, _D)


# speedup vs baseline: 1.5534x; 1.0001x over previous
import jax
import jax.numpy as jnp
from jax import lax
from jax.experimental import pallas as pl
from jax.experimental.pallas import tpu as pltpu
from jax.experimental.pallas import tpu_sc as plsc

_NC, _NS = 2, 16
_NW = _NC * _NS

_B, _H, _D, _V = 16384, 50, 32, 1000000
_SPW = _B // _NW
_LT = _SPW // 128
_DR = _D // 8


def _make_gather():
    mesh = plsc.VectorSubcoreMesh(
        core_axis_name="c", subcore_axis_name="s",
        num_cores=_NC, num_subcores=_NS,
    )

    def body(el_hbm, table_hbm, out_hbm, idx_v, idx_h, rows, tiles,
             g0, g1, w0, w1):
        gsem = (g0, g1)
        wsem = (w0, w1)
        wid = lax.axis_index("s") * _NC + lax.axis_index("c")
        base = wid * _SPW
        pltpu.sync_copy(el_hbm.at[pl.ds(base, _SPW)], idx_v)
        lanes = lax.iota(jnp.int32, 16)

        def build_idx(h, b):
            hvec = lanes * 0 + h
            for k in range(_SPW // 16):
                rowv = k * 16 + lanes
                idx_h[b, pl.ds(k * 16, 16)] = plsc.load_gather(
                    idx_v, [rowv, hvec]) * 4

        def gather_desc(b):
            return pltpu.make_async_copy(
                table_hbm.at[idx_h.at[b]], rows.at[b], gsem[b])

        def start_gather(b):
            pltpu.async_copy(table_hbm.at[idx_h.at[b]], rows.at[b], gsem[b])

        def out_slab(h):
            return out_hbm.at[pl.ds(h, 1), pl.ds(0, _DR), pl.ds(wid * _LT, _LT)]

        def tiles_view(b):
            return tiles.at[b, pl.ds(0, 1), pl.ds(0, _DR), pl.ds(0, _LT),
                            pl.ds(0, 8), pl.ds(0, 128)]

        def write_desc(b, h):
            return pltpu.make_async_copy(out_slab(h), tiles_view(b), wsem[b])

        dr_lo = lanes >> 3
        dr_hi = dr_lo + 2
        s_vec = lanes & 7
        zero16 = lanes * 0

        def transpose(b):
            def tj(jb, carry):
                jbase = jb * 16
                q_vec = zero16 + (jb >> 3)
                l_base = zero16 + ((jb & 7) * 16)
                for jj in range(16):
                    j = jbase + jj
                    l_vec = l_base + jj
                    v0 = rows[b, j, pl.ds(0, 16)]
                    v1 = rows[b, j, pl.ds(16, 16)]
                    plsc.store_scatter(
                        tiles.at[b, 0], [dr_lo, q_vec, s_vec, l_vec], v0)
                    plsc.store_scatter(
                        tiles.at[b, 0], [dr_hi, q_vec, s_vec, l_vec], v1)
                return carry
            lax.fori_loop(0, _SPW // 16, tj, 0)

        build_idx(0, 0)
        start_gather(0)
        build_idx(1, 1)
        start_gather(1)

        def step(hh, carry):
            for b in range(2):
                h = 2 * hh + b
                gather_desc(b).wait()

                @pl.when(hh > 0)
                def _():
                    write_desc(b, h - 2).wait()

                transpose(b)
                pltpu.async_copy(tiles_view(b), out_slab(h), wsem[b])

                @pl.when(h + 2 < _H)
                def _():
                    build_idx(h + 2, b)
                    start_gather(b)
            return carry

        lax.fori_loop(0, _H // 2, step, 0)
        write_desc(0, _H - 2).wait()
        write_desc(1, _H - 1).wait()

    return pl.kernel(
        body,
        out_type=jax.ShapeDtypeStruct((_H, _DR, 128, 8, 128), jnp.float32),
        mesh=mesh,
        scratch_types=[
            pltpu.VMEM((_SPW, _H), jnp.int32),
            pltpu.VMEM((2, _SPW), jnp.int32),
            pltpu.VMEM((2, _SPW, _D), jnp.float32),
            pltpu.VMEM((2, 1, _DR, _LT, 10, 129), jnp.float32),
            pltpu.SemaphoreType.DMA,
            pltpu.SemaphoreType.DMA,
            pltpu.SemaphoreType.DMA,
            pltpu.SemaphoreType.DMA,
        ],
        compiler_params=pltpu.CompilerParams(use_tc_tiling_on_sc=False, needs_layout_passes=False),
    )


def _tc_relayout(table):
    tt = table.T
    VB = 8192
    NBLK = 123
    NG = 62
    LAST = (_V - VB) // 128 * 128
    TCOL = _V // 128 * 128

    def body(x_hbm, o_hbm, xv0, xv1, ov0, ov1, xt, ot, si0, si1, so0, so1):
        g = pl.program_id(0)
        xv = (xv0, xv1)
        ov = (ov0, ov1)
        si = (si0, si1)
        so = (so0, so1)

        def col_of(i):
            return jnp.minimum(i * VB, LAST)

        def rd(i, k):
            return pltpu.make_async_copy(
                x_hbm.at[:, pl.ds(col_of(i), VB)], xv[k], si[k])

        def wr(i, k):
            return pltpu.make_async_copy(
                ov[k], o_hbm.at[pl.ds(col_of(i) // 8, VB // 8)], so[k])

        for k in range(2):
            rd(2 * g + k, k).start()
        for k in range(2):
            i = 2 * g + k
            rd(i, k).wait()

            @pl.when(g > 0)
            def _():
                wr(i - 2, k).wait()

            ov[k][:, :, pl.ds(0, _D)] = xv[k][...].T.reshape(VB // 8, 8, _D)
            wr(i, k).start()

        @pl.when(g == NG - 1)
        def _():
            for k in range(2):
                wr(2 * g + k, k).wait()
            ct = pltpu.make_async_copy(x_hbm.at[:, pl.ds(TCOL, 64)], xt, si0)
            ct.start()
            ct.wait()
            ot[:, :, pl.ds(0, _D)] = xt[...].T.reshape(8, 8, _D)
            cto = pltpu.make_async_copy(
                ot, o_hbm.at[pl.ds(TCOL // 8, 8)], so0)
            cto.start()
            cto.wait()

    out = pl.pallas_call(
        body,
        grid=(NG,),
        in_specs=[pl.BlockSpec(memory_space=pltpu.MemorySpace.HBM)],
        out_specs=pl.BlockSpec(memory_space=pltpu.MemorySpace.HBM),
        out_shape=jax.ShapeDtypeStruct((_V // 8, 8, 128), jnp.float32),
        scratch_shapes=[pltpu.VMEM((32, VB), jnp.float32),
                        pltpu.VMEM((32, VB), jnp.float32),
                        pltpu.VMEM((VB // 8, 8, 128), jnp.float32),
                        pltpu.VMEM((VB // 8, 8, 128), jnp.float32),
                        pltpu.VMEM((32, 64), jnp.float32),
                        pltpu.VMEM((8, 8, 128), jnp.float32),
                        pltpu.SemaphoreType.DMA,
                        pltpu.SemaphoreType.DMA,
                        pltpu.SemaphoreType.DMA,
                        pltpu.SemaphoreType.DMA],
    )(tt)
    return out.reshape(4 * _V, _D)


def kernel(element, table):
    out5 = _make_gather()(element, _tc_relayout(table))
    return out5.transpose((2, 4, 0, 1, 3)).reshape(_B, _H, _D)


# speedup vs baseline: 1.7286x; 1.1128x over previous
import jax
import jax.numpy as jnp
from jax import lax
from jax.experimental import pallas as pl
from jax.experimental.pallas import tpu as pltpu
from jax.experimental.pallas import tpu_sc as plsc

_NC, _NS = 2, 16
_NW = _NC * _NS

_B, _H, _D, _V = 16384, 50, 32, 1000000
_SPW = _B // _NW
_LT = _SPW // 128
_DR = _D // 8


def _make_gather():
    mesh = plsc.VectorSubcoreMesh(
        core_axis_name="c", subcore_axis_name="s",
        num_cores=_NC, num_subcores=_NS,
    )

    def body(el_hbm, table_hbm, out_hbm, idx_v, idx_h, rows, tiles,
             g0, g1, w0, w1):
        gsem = (g0, g1)
        wsem = (w0, w1)
        wid = lax.axis_index("s") * _NC + lax.axis_index("c")
        base = wid * _SPW
        pltpu.sync_copy(el_hbm.at[pl.ds(base, _SPW)], idx_v)
        lanes = lax.iota(jnp.int32, 16)

        def build_idx(h, b):
            hvec = lanes * 0 + h
            for k in range(_SPW // 16):
                rowv = k * 16 + lanes
                idx_h[b, pl.ds(k * 16, 16)] = plsc.load_gather(
                    idx_v, [rowv, hvec]) * 4

        def gather_desc(b):
            return pltpu.make_async_copy(
                table_hbm.at[idx_h.at[b]], rows.at[b], gsem[b])

        def start_gather(b):
            pltpu.async_copy(table_hbm.at[idx_h.at[b]], rows.at[b], gsem[b])

        def out_slab(h):
            return out_hbm.at[pl.ds(h, 1), pl.ds(0, _DR), pl.ds(wid * _LT, _LT)]

        def tiles_view(b):
            return tiles.at[b, pl.ds(0, 1), pl.ds(0, _DR), pl.ds(0, _LT),
                            pl.ds(0, 8), pl.ds(0, 128)]

        def write_desc(b, h):
            return pltpu.make_async_copy(out_slab(h), tiles_view(b), wsem[b])

        dr_lo = lanes >> 3
        dr_hi = dr_lo + 2
        s_vec = lanes & 7
        zero16 = lanes * 0

        def transpose(b):
            def tj(jb, carry):
                jbase = jb * 16
                q_vec = zero16 + (jb >> 3)
                l_base = zero16 + ((jb & 7) * 16)
                for jj in range(16):
                    j = jbase + jj
                    l_vec = l_base + jj
                    v0 = rows[b, j, pl.ds(0, 16)]
                    v1 = rows[b, j, pl.ds(16, 16)]
                    plsc.store_scatter(
                        tiles.at[b, 0], [dr_lo, q_vec, s_vec, l_vec], v0)
                    plsc.store_scatter(
                        tiles.at[b, 0], [dr_hi, q_vec, s_vec, l_vec], v1)
                return carry
            lax.fori_loop(0, _SPW // 16, tj, 0)

        build_idx(0, 0)
        start_gather(0)
        build_idx(1, 1)
        start_gather(1)

        def step(hh, carry):
            for b in range(2):
                h = 2 * hh + b
                gather_desc(b).wait()

                @pl.when(hh > 0)
                def _():
                    write_desc(b, h - 2).wait()

                transpose(b)
                pltpu.async_copy(tiles_view(b), out_slab(h), wsem[b])

                @pl.when(h + 2 < _H)
                def _():
                    build_idx(h + 2, b)
                    start_gather(b)
            return carry

        lax.fori_loop(0, _H // 2, step, 0)
        write_desc(0, _H - 2).wait()
        write_desc(1, _H - 1).wait()

    return pl.kernel(
        body,
        out_type=jax.ShapeDtypeStruct((_H, _DR, 128, 8, 128), jnp.float32),
        mesh=mesh,
        scratch_types=[
            pltpu.VMEM((_SPW, _H), jnp.int32),
            pltpu.VMEM((2, _SPW), jnp.int32),
            pltpu.VMEM((2, _SPW, _D), jnp.float32),
            pltpu.VMEM((2, 1, _DR, _LT, 10, 129), jnp.float32),
            pltpu.SemaphoreType.DMA,
            pltpu.SemaphoreType.DMA,
            pltpu.SemaphoreType.DMA,
            pltpu.SemaphoreType.DMA,
        ],
        compiler_params=pltpu.CompilerParams(use_tc_tiling_on_sc=False, needs_layout_passes=False),
    )


def _tc_relayout(table):
    tt = table.T
    VB = 16384
    NG = 31
    LAST = (_V - VB) // 128 * 128
    TCOL = _V // 128 * 128

    def body(x_hbm, o_hbm, xv0, xv1, ov0, ov1, xt, ot, si0, si1, so0, so1):
        g = pl.program_id(0)
        xv = (xv0, xv1)
        ov = (ov0, ov1)
        si = (si0, si1)
        so = (so0, so1)

        def col_of(i):
            return jnp.minimum(i * VB, LAST)

        def rd(i, k):
            return pltpu.make_async_copy(
                x_hbm.at[:, pl.ds(col_of(i), VB)], xv[k], si[k])

        def wr(i, k):
            return pltpu.make_async_copy(
                ov[k], o_hbm.at[pl.ds(col_of(i) // 8, VB // 8)], so[k])

        for k in range(2):
            rd(2 * g + k, k).start()
        for k in range(2):
            i = 2 * g + k
            rd(i, k).wait()

            @pl.when(g > 0)
            def _():
                wr(i - 2, k).wait()

            ov[k][:, :, pl.ds(0, _D)] = xv[k][...].T.reshape(VB // 8, 8, _D)
            wr(i, k).start()

        @pl.when(g == NG - 1)
        def _():
            for k in range(2):
                wr(2 * g + k, k).wait()
            ct = pltpu.make_async_copy(x_hbm.at[:, pl.ds(TCOL, 64)], xt, si0)
            ct.start()
            ct.wait()
            ot[:, :, pl.ds(0, _D)] = xt[...].T.reshape(8, 8, _D)
            cto = pltpu.make_async_copy(
                ot, o_hbm.at[pl.ds(TCOL // 8, 8)], so0)
            cto.start()
            cto.wait()

    out = pl.pallas_call(
        body,
        grid=(NG,),
        in_specs=[pl.BlockSpec(memory_space=pltpu.MemorySpace.HBM)],
        out_specs=pl.BlockSpec(memory_space=pltpu.MemorySpace.HBM),
        out_shape=jax.ShapeDtypeStruct((_V // 8, 8, 128), jnp.float32),
        scratch_shapes=[pltpu.VMEM((32, VB), jnp.float32),
                        pltpu.VMEM((32, VB), jnp.float32),
                        pltpu.VMEM((VB // 8, 8, 128), jnp.float32),
                        pltpu.VMEM((VB // 8, 8, 128), jnp.float32),
                        pltpu.VMEM((32, 64), jnp.float32),
                        pltpu.VMEM((8, 8, 128), jnp.float32),
                        pltpu.SemaphoreType.DMA,
                        pltpu.SemaphoreType.DMA,
                        pltpu.SemaphoreType.DMA,
                        pltpu.SemaphoreType.DMA],
    )(tt)
    return out.reshape(4 * _V, _D)


def kernel(element, table):
    out5 = _make_gather()(element, _tc_relayout(table))
    return out5.transpose((2, 4, 0, 1, 3)).reshape(_B, _H, _D)


# speedup vs baseline: 1.8018x; 1.0424x over previous
import jax
import jax.numpy as jnp
from jax import lax
from jax.experimental import pallas as pl
from jax.experimental.pallas import tpu as pltpu
from jax.experimental.pallas import tpu_sc as plsc

_NC, _NS = 2, 16
_NW = _NC * _NS

_B, _H, _D, _V = 16384, 50, 32, 1000000
_SPW = _B // _NW
_LT = _SPW // 128
_DR = _D // 8


def _make_gather():
    mesh = plsc.VectorSubcoreMesh(
        core_axis_name="c", subcore_axis_name="s",
        num_cores=_NC, num_subcores=_NS,
    )

    def body(el_hbm, table_hbm, out_hbm, idx_v, idx_h, rows, tiles,
             g0, g1, w0, w1):
        gsem = (g0, g1)
        wsem = (w0, w1)
        wid = lax.axis_index("s") * _NC + lax.axis_index("c")
        base = wid * _SPW
        pltpu.sync_copy(el_hbm.at[pl.ds(base, _SPW)], idx_v)
        lanes = lax.iota(jnp.int32, 16)

        def build_idx(h, b):
            hvec = lanes * 0 + h
            for k in range(_SPW // 16):
                rowv = k * 16 + lanes
                idx_h[b, pl.ds(k * 16, 16)] = plsc.load_gather(
                    idx_v, [rowv, hvec]) * 4

        def gather_desc(b):
            return pltpu.make_async_copy(
                table_hbm.at[idx_h.at[b]], rows.at[b], gsem[b])

        def start_gather(b):
            pltpu.async_copy(table_hbm.at[idx_h.at[b]], rows.at[b], gsem[b])

        def out_slab(h):
            return out_hbm.at[pl.ds(h, 1), pl.ds(0, _DR), pl.ds(wid * _LT, _LT)]

        def tiles_view(b):
            return tiles.at[b, pl.ds(0, 1), pl.ds(0, _DR), pl.ds(0, _LT),
                            pl.ds(0, 8), pl.ds(0, 128)]

        def write_desc(b, h):
            return pltpu.make_async_copy(out_slab(h), tiles_view(b), wsem[b])

        dr_lo = lanes >> 3
        dr_hi = dr_lo + 2
        s_vec = lanes & 7
        zero16 = lanes * 0

        def transpose(b):
            def tj(jb, carry):
                jbase = jb * 16
                q_vec = zero16 + (jb >> 3)
                l_base = zero16 + ((jb & 7) * 16)
                for jj in range(16):
                    j = jbase + jj
                    l_vec = l_base + jj
                    v0 = rows[b, j, pl.ds(0, 16)]
                    v1 = rows[b, j, pl.ds(16, 16)]
                    plsc.store_scatter(
                        tiles.at[b, 0], [dr_lo, q_vec, s_vec, l_vec], v0)
                    plsc.store_scatter(
                        tiles.at[b, 0], [dr_hi, q_vec, s_vec, l_vec], v1)
                return carry
            lax.fori_loop(0, _SPW // 16, tj, 0)

        build_idx(0, 0)
        start_gather(0)
        build_idx(1, 1)
        start_gather(1)

        def step(hh, carry):
            for b in range(2):
                h = 2 * hh + b
                gather_desc(b).wait()

                @pl.when(hh > 0)
                def _():
                    write_desc(b, h - 2).wait()

                transpose(b)
                pltpu.async_copy(tiles_view(b), out_slab(h), wsem[b])

                @pl.when(h + 2 < _H)
                def _():
                    build_idx(h + 2, b)
                    start_gather(b)
            return carry

        lax.fori_loop(0, _H // 2, step, 0)
        write_desc(0, _H - 2).wait()
        write_desc(1, _H - 1).wait()

    return pl.kernel(
        body,
        out_type=jax.ShapeDtypeStruct((_H, _DR, 128, 8, 128), jnp.float32),
        mesh=mesh,
        scratch_types=[
            pltpu.VMEM((_SPW, _H), jnp.int32),
            pltpu.VMEM((2, _SPW), jnp.int32),
            pltpu.VMEM((2, _SPW, _D), jnp.float32),
            pltpu.VMEM((2, 1, _DR, _LT, 10, 129), jnp.float32),
            pltpu.SemaphoreType.DMA,
            pltpu.SemaphoreType.DMA,
            pltpu.SemaphoreType.DMA,
            pltpu.SemaphoreType.DMA,
        ],
        compiler_params=pltpu.CompilerParams(use_tc_tiling_on_sc=False, needs_layout_passes=False),
    )


def _tc_relayout(table):
    tt = table.T
    VB = 32768
    NG = 16
    LAST = (_V - VB) // 128 * 128
    TCOL = _V // 128 * 128

    def body(x_hbm, o_hbm, xv0, xv1, ov0, ov1, xt, ot, si0, si1, so0, so1):
        g = pl.program_id(0)
        xv = (xv0, xv1)
        ov = (ov0, ov1)
        si = (si0, si1)
        so = (so0, so1)

        def col_of(i):
            return jnp.minimum(i * VB, LAST)

        def rd(i, k):
            return pltpu.make_async_copy(
                x_hbm.at[:, pl.ds(col_of(i), VB)], xv[k], si[k])

        def wr(i, k):
            return pltpu.make_async_copy(
                ov[k], o_hbm.at[pl.ds(col_of(i) // 8, VB // 8)], so[k])

        for k in range(2):
            rd(2 * g + k, k).start()
        for k in range(2):
            i = 2 * g + k
            rd(i, k).wait()

            @pl.when(g > 0)
            def _():
                wr(i - 2, k).wait()

            ov[k][:, :, pl.ds(0, _D)] = xv[k][...].T.reshape(VB // 8, 8, _D)
            wr(i, k).start()

        @pl.when(g == NG - 1)
        def _():
            for k in range(2):
                wr(2 * g + k, k).wait()
            ct = pltpu.make_async_copy(x_hbm.at[:, pl.ds(TCOL, 64)], xt, si0)
            ct.start()
            ct.wait()
            ot[:, :, pl.ds(0, _D)] = xt[...].T.reshape(8, 8, _D)
            cto = pltpu.make_async_copy(
                ot, o_hbm.at[pl.ds(TCOL // 8, 8)], so0)
            cto.start()
            cto.wait()

    out = pl.pallas_call(
        body,
        grid=(NG,),
        in_specs=[pl.BlockSpec(memory_space=pltpu.MemorySpace.HBM)],
        out_specs=pl.BlockSpec(memory_space=pltpu.MemorySpace.HBM),
        out_shape=jax.ShapeDtypeStruct((_V // 8, 8, 128), jnp.float32),
        scratch_shapes=[pltpu.VMEM((32, VB), jnp.float32),
                        pltpu.VMEM((32, VB), jnp.float32),
                        pltpu.VMEM((VB // 8, 8, 128), jnp.float32),
                        pltpu.VMEM((VB // 8, 8, 128), jnp.float32),
                        pltpu.VMEM((32, 64), jnp.float32),
                        pltpu.VMEM((8, 8, 128), jnp.float32),
                        pltpu.SemaphoreType.DMA,
                        pltpu.SemaphoreType.DMA,
                        pltpu.SemaphoreType.DMA,
                        pltpu.SemaphoreType.DMA],
    )(tt)
    return out.reshape(4 * _V, _D)


def kernel(element, table):
    out5 = _make_gather()(element, _tc_relayout(table))
    return out5.transpose((2, 4, 0, 1, 3)).reshape(_B, _H, _D)
